# baseline (speedup 1.0000x reference)
import functools

import jax
import jax.numpy as jnp
from jax import lax
from jax.experimental import pallas as pl
from jax.experimental.pallas import tpu as pltpu
from jax.experimental.pallas import tpu_sc as plsc

_K = 51
_LN2 = 0.6931471805599453


def _inv_sqrt(x):
    bits = lax.bitcast_convert_type(x, jnp.int32)
    y = lax.bitcast_convert_type(
        jnp.int32(0x5F3759DF) - (bits >> 1), jnp.float32)
    for _ in range(3):
        y = y * (1.5 - 0.5 * x * y * y)
    return y


def _log(x):
    bits = lax.bitcast_convert_type(x, jnp.int32)
    e = (bits >> 23) - 127
    m = lax.bitcast_convert_type(
        (bits & jnp.int32(0x007FFFFF)) | jnp.int32(0x3F800000), jnp.float32)
    t = (m - 1.0) / (m + 1.0)
    t2 = t * t
    p = 1.0 + t2 * (1.0 / 3.0 + t2 * (1.0 / 5.0 + t2 * (1.0 / 7.0 + t2 / 9.0)))
    return e.astype(jnp.float32) * _LN2 + 2.0 * t * p


def _make_sc_kernel(B, V, D):
    try:
        info = plsc.get_sparse_core_info()
        NC, NS, L = info.num_cores, info.num_subcores, info.num_lanes
    except ValueError:
        NC, NS, L = 2, 16, 16
    NW = NC * NS
    bpw = B // NW
    GB = 8
    NG = bpw // GB

    mesh = plsc.VectorSubcoreMesh(
        core_axis_name="c", subcore_axis_name="s",
        num_cores=NC, num_subcores=NS)

    @functools.partial(
        pl.kernel,
        out_type=[
            jax.ShapeDtypeStruct((B,), jnp.float32),
            jax.ShapeDtypeStruct((B,), jnp.float32),
        ],
        mesh=mesh,
        scratch_types=[
            pltpu.VMEM((bpw // 2, 2 * D), jnp.int32),
            pltpu.VMEM((bpw // 2, 2 * D), jnp.float32),
            pltpu.VMEM((GB, 4 * L, 2 * D), jnp.float32),
            pltpu.VMEM((GB, 4 * L), jnp.int32),
            [pltpu.SemaphoreType.DMA for _ in range(GB)],
            pltpu.VMEM((bpw,), jnp.float32),
            pltpu.VMEM((bpw,), jnp.float32),
        ],
        compiler_params=pltpu.CompilerParams(
            needs_layout_passes=False, use_tc_tiling_on_sc=True),
    )
    def sc_kernel(bnns_hbm, recv_hbm, emb_hbm, loss_hbm, acc_hbm,
                  ids_v, recv_v, rows_v, par_v, rsems, loss_v, acc_v):
        wid = lax.axis_index("s") * NC + lax.axis_index("c")
        base = wid * bpw
        hrows = bpw // 2
        hbase = wid * hrows
        lane = lax.iota(jnp.int32, L)
        kmask3 = lane < (_K - 3 * L)

        pltpu.sync_copy(bnns_hbm.at[pl.ds(hbase, hrows)], ids_v)
        pltpu.sync_copy(recv_hbm.at[pl.ds(hbase, hrows)], recv_v)

        def hop2_issue(g, bi):
            rowv = jnp.full((L,), g * (GB // 2) + bi // 2, jnp.int32)
            cb = (bi & 1) * D
            for kc in range(4):
                ids = plsc.load_gather(ids_v, [rowv, cb + kc * L + lane])
                par_v[bi, pl.ds(kc * L, L)] = ids & 1
                pltpu.make_async_copy(
                    emb_hbm.at[ids >> 1],
                    rows_v.at[bi, pl.ds(kc * L, L)], rsems[bi]).start()

        def rows_wait(bi):
            for kc in range(4):
                pltpu.make_async_copy(
                    emb_hbm.at[lane],
                    rows_v.at[bi, pl.ds(kc * L, L)], rsems[bi]).wait()

        def compute(g, bi):
            rrow = jnp.full((L,), g * (GB // 2) + bi // 2, jnp.int32)
            cb = (bi & 1) * D
            r = [plsc.load_gather(recv_v, [rrow, cb + c * L + lane])
                 for c in range(D // L)]
            norm2 = jnp.sum(sum(rc * rc for rc in r))
            inv_norm = _inv_sqrt(norm2)
            biv = jnp.full((L,), bi, jnp.int32)
            colb = [par_v[bi, pl.ds(kc * L, L)] * D for kc in range(4)]

            def dbody(dj, acc):
                out = list(acc)
                for i in range(8):
                    d = dj * 8 + i
                    rd = plsc.load_gather(
                        recv_v, [rrow, jnp.full((L,), cb + d, jnp.int32)])
                    for kc in range(4):
                        kidx = lane + kc * L
                        mask = kmask3 if kc == 3 else None
                        vals = plsc.load_gather(
                            rows_v, [biv, kidx, colb[kc] + d], mask=mask)
                        out[kc] = out[kc] + vals * rd
                return tuple(out)

            zero = jnp.zeros((L,), jnp.float32)
            acc = lax.fori_loop(0, D // 8, dbody, (zero, zero, zero, zero))

            cos = [a * inv_norm for a in acc]
            cos3 = jnp.where(kmask3, cos[3], -1e30)
            m = jnp.max(jnp.maximum(jnp.maximum(cos[0], cos[1]),
                                    jnp.maximum(cos[2], cos3)))
            es = [jnp.exp(c - m) for c in cos[:3]]
            e3 = jnp.where(kmask3, jnp.exp(cos3 - m), 0.0)
            s = jnp.sum(es[0] + es[1] + es[2] + e3)
            logs = _log(jnp.full((L,), s, jnp.float32))
            loss = logs[0] + m - cos[0][0]

            rmax = jnp.max(jnp.maximum(jnp.maximum(r[0], r[1]),
                                       jnp.maximum(r[2], r[3])))
            accv = (r[0][0] >= rmax).astype(jnp.float32)
            return loss, accv

        def gbody(g, carry):
            lv, av = carry
            for bi in range(GB):
                hop2_issue(g, bi)
            for bi in range(GB):
                rows_wait(bi)
                loss, accv = compute(g, bi)
                sel = lane == ((g & 1) * GB + bi)
                lv = jnp.where(sel, loss, lv)
                av = jnp.where(sel, accv, av)

            @pl.when((g & 1) == 1)
            def _():
                o = pl.multiple_of((g // 2) * L, L)
                loss_v[pl.ds(o, L)] = lv
                acc_v[pl.ds(o, L)] = av

            return lv, av

        zero = jnp.zeros((L,), jnp.float32)
        lax.fori_loop(0, NG, gbody, (zero, zero))

        pltpu.sync_copy(loss_v, loss_hbm.at[pl.ds(base, bpw)])
        pltpu.sync_copy(acc_v, acc_hbm.at[pl.ds(base, bpw)])

    return sc_kernel


def kernel(_sender_input, _message, _receiver_input, receiver_output,
           _labels, _aux_input, train_emb, train_nns, dev_emb, dev_nns):
    B, D = receiver_output.shape
    V, NN = train_nns.shape
    batch_nns = jnp.take(train_nns, _labels, axis=0)
    batch_nns = jnp.concatenate(
        [batch_nns, jnp.zeros((B, D - NN), jnp.int32)], axis=1)
    bnns2 = batch_nns.reshape(B // 2, 2 * D)
    recv2 = receiver_output.reshape(B // 2, 2 * D)
    emb2 = train_emb.reshape(V // 2, 2 * D)
    sc = _make_sc_kernel(B, V, D)
    loss, acc = sc(bnns2, recv2, emb2)
    return (loss, acc)


# speedup vs baseline: 1.6124x; 1.6124x over previous
import functools

import jax
import jax.numpy as jnp
from jax import lax
from jax.experimental import pallas as pl
from jax.experimental.pallas import tpu as pltpu
from jax.experimental.pallas import tpu_sc as plsc

_K = 51
_LN2 = 0.6931471805599453


def _inv_sqrt(x):
    bits = lax.bitcast_convert_type(x, jnp.int32)
    y = lax.bitcast_convert_type(
        jnp.int32(0x5F3759DF) - (bits >> 1), jnp.float32)
    for _ in range(3):
        y = y * (1.5 - 0.5 * x * y * y)
    return y


def _log(x):
    bits = lax.bitcast_convert_type(x, jnp.int32)
    e = (bits >> 23) - 127
    m = lax.bitcast_convert_type(
        (bits & jnp.int32(0x007FFFFF)) | jnp.int32(0x3F800000), jnp.float32)
    t = (m - 1.0) / (m + 1.0)
    t2 = t * t
    p = 1.0 + t2 * (1.0 / 3.0 + t2 * (1.0 / 5.0 + t2 * (1.0 / 7.0 + t2 / 9.0)))
    return e.astype(jnp.float32) * _LN2 + 2.0 * t * p


def _make_sc_kernel(B, V, D):
    try:
        info = plsc.get_sparse_core_info()
        NC, NS, L = info.num_cores, info.num_subcores, info.num_lanes
    except ValueError:
        NC, NS, L = 2, 16, 16
    NW = NC * NS
    bpw = B // NW
    NG = bpw // L

    mesh = plsc.VectorSubcoreMesh(
        core_axis_name="c", subcore_axis_name="s",
        num_cores=NC, num_subcores=NS)

    @functools.partial(
        pl.kernel,
        out_type=[
            jax.ShapeDtypeStruct((B,), jnp.float32),
            jax.ShapeDtypeStruct((B,), jnp.float32),
        ],
        mesh=mesh,
        scratch_types=[
            pltpu.VMEM((bpw, D), jnp.int32),
            pltpu.VMEM((bpw, D), jnp.float32),
            pltpu.VMEM((L, 4 * L, D), jnp.float32),
            [pltpu.SemaphoreType.DMA for _ in range(L)],
            pltpu.VMEM((bpw,), jnp.float32),
            pltpu.VMEM((bpw,), jnp.float32),
        ],
        compiler_params=pltpu.CompilerParams(
            needs_layout_passes=False, use_tc_tiling_on_sc=False),
    )
    def sc_kernel(bnns_hbm, recv_hbm, emb_hbm, loss_hbm, acc_hbm,
                  ids_v, recv_v, rows_v, rsems, loss_v, acc_v):
        wid = lax.axis_index("s") * NC + lax.axis_index("c")
        base = wid * bpw
        lane = lax.iota(jnp.int32, L)
        kmask3 = lane < (_K - 3 * L)

        pltpu.sync_copy(bnns_hbm.at[pl.ds(base, bpw)], ids_v)
        pltpu.sync_copy(recv_hbm.at[pl.ds(base, bpw)], recv_v)

        def hop2_issue(b, bi):
            for kc in range(4):
                ids = ids_v[b, pl.ds(kc * L, L)]
                pltpu.make_async_copy(
                    emb_hbm.at[ids],
                    rows_v.at[bi, pl.ds(kc * L, L)], rsems[bi]).start()

        def rows_wait(bi):
            for kc in range(4):
                pltpu.make_async_copy(
                    emb_hbm.at[lane],
                    rows_v.at[bi, pl.ds(kc * L, L)], rsems[bi]).wait()

        def compute(b, bi):
            r = [recv_v[b, pl.ds(c * L, L)] for c in range(D // L)]
            norm2 = jnp.sum(sum(rc * rc for rc in r))
            inv_norm = _inv_sqrt(norm2)
            biv = jnp.full((L,), bi, jnp.int32)
            bv = jnp.full((L,), b, jnp.int32)

            def dbody(dj, acc):
                out = list(acc)
                for i in range(8):
                    d = dj * 8 + i
                    rd = plsc.load_gather(
                        recv_v, [bv, jnp.full((L,), d, jnp.int32)])
                    dvec = jnp.full((L,), d, jnp.int32)
                    for kc in range(4):
                        kidx = lane + kc * L
                        mask = kmask3 if kc == 3 else None
                        vals = plsc.load_gather(
                            rows_v, [biv, kidx, dvec], mask=mask)
                        out[kc] = out[kc] + vals * rd
                return tuple(out)

            zero = jnp.zeros((L,), jnp.float32)
            acc = lax.fori_loop(0, D // 8, dbody, (zero, zero, zero, zero))

            cos = [a * inv_norm for a in acc]
            cos3 = jnp.where(kmask3, cos[3], -1e30)
            m = jnp.max(jnp.maximum(jnp.maximum(cos[0], cos[1]),
                                    jnp.maximum(cos[2], cos3)))
            es = [jnp.exp(c - m) for c in cos[:3]]
            e3 = jnp.where(kmask3, jnp.exp(cos3 - m), 0.0)
            s = jnp.sum(es[0] + es[1] + es[2] + e3)
            logs = _log(jnp.full((L,), s, jnp.float32))
            loss = logs[0] + m - cos[0][0]

            rmax = jnp.max(jnp.maximum(jnp.maximum(r[0], r[1]),
                                       jnp.maximum(r[2], r[3])))
            accv = (r[0][0] >= rmax).astype(jnp.float32)
            return loss, accv

        def gbody(g, carry):
            for bi in range(L):
                hop2_issue(g * L + bi, bi)
            lv = jnp.zeros((L,), jnp.float32)
            av = jnp.zeros((L,), jnp.float32)
            for bi in range(L):
                rows_wait(bi)
                loss, accv = compute(g * L + bi, bi)
                sel = lane == bi
                lv = jnp.where(sel, loss, lv)
                av = jnp.where(sel, accv, av)

            o = pl.multiple_of(g * L, L)
            loss_v[pl.ds(o, L)] = lv
            acc_v[pl.ds(o, L)] = av
            return carry

        lax.fori_loop(0, NG, gbody, 0)

        pltpu.sync_copy(loss_v, loss_hbm.at[pl.ds(base, bpw)])
        pltpu.sync_copy(acc_v, acc_hbm.at[pl.ds(base, bpw)])

    return sc_kernel


def kernel(_sender_input, _message, _receiver_input, receiver_output,
           _labels, _aux_input, train_emb, train_nns, dev_emb, dev_nns):
    B, D = receiver_output.shape
    V, NN = train_nns.shape
    batch_nns = jnp.take(train_nns, _labels, axis=0)
    batch_nns = jnp.concatenate(
        [batch_nns, jnp.zeros((B, D - NN), jnp.int32)], axis=1)
    sc = _make_sc_kernel(B, V, D)
    loss, acc = sc(batch_nns, receiver_output, train_emb)
    return (loss, acc)


# speedup vs baseline: 3.0159x; 1.8704x over previous
import functools

import jax
import jax.numpy as jnp
from jax import lax
from jax.experimental import pallas as pl
from jax.experimental.pallas import tpu as pltpu
from jax.experimental.pallas import tpu_sc as plsc

_K = 51
_LN2 = 0.6931471805599453


def _inv_sqrt(x):
    bits = lax.bitcast_convert_type(x, jnp.int32)
    y = lax.bitcast_convert_type(
        jnp.int32(0x5F3759DF) - (bits >> 1), jnp.float32)
    for _ in range(3):
        y = y * (1.5 - 0.5 * x * y * y)
    return y


def _log(x):
    bits = lax.bitcast_convert_type(x, jnp.int32)
    e = (bits >> 23) - 127
    m = lax.bitcast_convert_type(
        (bits & jnp.int32(0x007FFFFF)) | jnp.int32(0x3F800000), jnp.float32)
    t = (m - 1.0) / (m + 1.0)
    t2 = t * t
    p = 1.0 + t2 * (1.0 / 3.0 + t2 * (1.0 / 5.0 + t2 * (1.0 / 7.0 + t2 / 9.0)))
    return e.astype(jnp.float32) * _LN2 + 2.0 * t * p


def _make_sc_kernel(B, V, D):
    try:
        info = plsc.get_sparse_core_info()
        NC, NS, L = info.num_cores, info.num_subcores, info.num_lanes
    except ValueError:
        NC, NS, L = 2, 16, 16
    NW = NC * NS
    bpw = B // NW
    NG = bpw // L

    mesh = plsc.VectorSubcoreMesh(
        core_axis_name="c", subcore_axis_name="s",
        num_cores=NC, num_subcores=NS)

    @functools.partial(
        pl.kernel,
        out_type=[
            jax.ShapeDtypeStruct((B,), jnp.float32),
            jax.ShapeDtypeStruct((B,), jnp.float32),
        ],
        mesh=mesh,
        scratch_types=[
            pltpu.VMEM((bpw, D), jnp.int32),
            pltpu.VMEM((bpw, D), jnp.float32),
            pltpu.VMEM((L, 4 * L, D), jnp.float32),
            [pltpu.SemaphoreType.DMA for _ in range(L)],
            pltpu.VMEM((bpw,), jnp.float32),
            pltpu.VMEM((bpw,), jnp.float32),
        ],
        compiler_params=pltpu.CompilerParams(
            needs_layout_passes=False, use_tc_tiling_on_sc=False),
    )
    def sc_kernel(bnns_hbm, recv_hbm, emb_hbm, loss_hbm, acc_hbm,
                  ids_v, recv_v, rows_v, rsems, loss_v, acc_v):
        wid = lax.axis_index("s") * NC + lax.axis_index("c")
        base = wid * bpw
        lane = lax.iota(jnp.int32, L)
        kmask3 = lane < (_K - 3 * L)

        pltpu.sync_copy(bnns_hbm.at[pl.ds(base, bpw)], ids_v)
        pltpu.sync_copy(recv_hbm.at[pl.ds(base, bpw)], recv_v)

        def hop2_issue(b, bi):
            for kc in range(4):
                ids = ids_v[b, pl.ds(kc * L, L)]
                pltpu.make_async_copy(
                    emb_hbm.at[ids],
                    rows_v.at[bi, pl.ds(kc * L, L)], rsems[bi]).start()

        def rows_wait(bi):
            for kc in range(4):
                pltpu.make_async_copy(
                    emb_hbm.at[lane],
                    rows_v.at[bi, pl.ds(kc * L, L)], rsems[bi]).wait()

        def compute(b, bi):
            r = [recv_v[b, pl.ds(c * L, L)] for c in range(D // L)]
            norm2 = jnp.sum(sum(rc * rc for rc in r))
            inv_norm = _inv_sqrt(norm2)
            biv = jnp.full((L,), bi, jnp.int32)
            bv = jnp.full((L,), b, jnp.int32)

            def dbody(dj, acc):
                out = list(acc)
                for i in range(8):
                    d = dj * 8 + i
                    rd = plsc.load_gather(
                        recv_v, [bv, jnp.full((L,), d, jnp.int32)])
                    dvec = jnp.full((L,), d, jnp.int32)
                    for kc in range(4):
                        kidx = lane + kc * L
                        mask = kmask3 if kc == 3 else None
                        vals = plsc.load_gather(
                            rows_v, [biv, kidx, dvec], mask=mask)
                        out[kc] = out[kc] + vals * rd
                return tuple(out)

            zero = jnp.zeros((L,), jnp.float32)
            acc = lax.fori_loop(0, D // 8, dbody, (zero, zero, zero, zero))

            cos = [a * inv_norm for a in acc]
            cos3 = jnp.where(kmask3, cos[3], -1e30)
            m = jnp.max(jnp.maximum(jnp.maximum(cos[0], cos[1]),
                                    jnp.maximum(cos[2], cos3)))
            es = [jnp.exp(c - m) for c in cos[:3]]
            e3 = jnp.where(kmask3, jnp.exp(cos3 - m), 0.0)
            s = jnp.sum(es[0] + es[1] + es[2] + e3)
            logs = _log(jnp.full((L,), s, jnp.float32))
            loss = logs[0] + m - cos[0][0]

            rmax = jnp.max(jnp.maximum(jnp.maximum(r[0], r[1]),
                                       jnp.maximum(r[2], r[3])))
            accv = (r[0][0] >= rmax).astype(jnp.float32)
            return loss, accv

        def gbody(g, carry):
            for bi in range(L):
                hop2_issue(g * L + bi, bi)
            lv = jnp.zeros((L,), jnp.float32)
            av = jnp.zeros((L,), jnp.float32)
            for bi in range(L):
                rows_wait(bi)
                loss, accv = compute(g * L + bi, bi)
                sel = lane == bi
                lv = jnp.where(sel, loss, lv)
                av = jnp.where(sel, accv, av)

            o = pl.multiple_of(g * L, L)
            loss_v[pl.ds(o, L)] = lv
            acc_v[pl.ds(o, L)] = av
            return carry

        lax.fori_loop(0, NG, gbody, 0)

        pltpu.sync_copy(loss_v, loss_hbm.at[pl.ds(base, bpw)])
        pltpu.sync_copy(acc_v, acc_hbm.at[pl.ds(base, bpw)])

    return sc_kernel


def kernel(_sender_input, _message, _receiver_input, receiver_output,
           _labels, _aux_input, train_emb, train_nns, dev_emb, dev_nns):
    B, D = receiver_output.shape
    V, NN = train_nns.shape
    batch_nns = jnp.take(train_nns, _labels, axis=0)
    batch_nns = jnp.concatenate(
        [batch_nns, batch_nns[:, : D - NN]], axis=1)
    sc = _make_sc_kernel(B, V, D)
    loss, acc = sc(batch_nns, receiver_output, train_emb)
    return (loss, acc)


# speedup vs baseline: 3.8459x; 1.2752x over previous
import functools

import jax
import jax.numpy as jnp
from jax import lax
from jax.experimental import pallas as pl
from jax.experimental.pallas import tpu as pltpu
from jax.experimental.pallas import tpu_sc as plsc

_K = 51
_LN2 = 0.6931471805599453


def _inv_sqrt(x):
    bits = lax.bitcast_convert_type(x, jnp.int32)
    y = lax.bitcast_convert_type(
        jnp.int32(0x5F3759DF) - (bits >> 1), jnp.float32)
    for _ in range(3):
        y = y * (1.5 - 0.5 * x * y * y)
    return y


def _log(x):
    bits = lax.bitcast_convert_type(x, jnp.int32)
    e = (bits >> 23) - 127
    m = lax.bitcast_convert_type(
        (bits & jnp.int32(0x007FFFFF)) | jnp.int32(0x3F800000), jnp.float32)
    t = (m - 1.0) / (m + 1.0)
    t2 = t * t
    p = 1.0 + t2 * (1.0 / 3.0 + t2 * (1.0 / 5.0 + t2 * (1.0 / 7.0 + t2 / 9.0)))
    return e.astype(jnp.float32) * _LN2 + 2.0 * t * p


def _make_sc_kernel(B, V, D):
    try:
        info = plsc.get_sparse_core_info()
        NC, NS, L = info.num_cores, info.num_subcores, info.num_lanes
    except ValueError:
        NC, NS, L = 2, 16, 16
    NW = NC * NS
    bpw = B // NW
    NG = bpw // L

    mesh = plsc.VectorSubcoreMesh(
        core_axis_name="c", subcore_axis_name="s",
        num_cores=NC, num_subcores=NS)

    @functools.partial(
        pl.kernel,
        out_type=[
            jax.ShapeDtypeStruct((B,), jnp.float32),
            jax.ShapeDtypeStruct((B,), jnp.float32),
        ],
        mesh=mesh,
        scratch_types=[
            pltpu.VMEM((bpw, D), jnp.int32),
            pltpu.VMEM((bpw, D), jnp.float32),
            pltpu.VMEM((L, 4 * L, D), jnp.float32),
            [pltpu.SemaphoreType.DMA for _ in range(L)],
            pltpu.VMEM((bpw,), jnp.float32),
            pltpu.VMEM((bpw,), jnp.float32),
        ],
        compiler_params=pltpu.CompilerParams(
            needs_layout_passes=False, use_tc_tiling_on_sc=False),
    )
    def sc_kernel(bnns_hbm, recv_hbm, emb_hbm, loss_hbm, acc_hbm,
                  ids_v, recv_v, rows_v, rsems, loss_v, acc_v):
        wid = lax.axis_index("s") * NC + lax.axis_index("c")
        base = wid * bpw
        lane = lax.iota(jnp.int32, L)
        kmask3 = lane < (_K - 3 * L)

        pltpu.sync_copy(bnns_hbm.at[pl.ds(base, bpw)], ids_v)
        pltpu.sync_copy(recv_hbm.at[pl.ds(base, bpw)], recv_v)

        def hop2_issue(b, bi):
            for kc in range(4):
                ids = ids_v[b, pl.ds(kc * L, L)]
                pltpu.make_async_copy(
                    emb_hbm.at[ids],
                    rows_v.at[bi, pl.ds(kc * L, L)], rsems[bi]).start()

        def rows_wait(bi):
            for kc in range(4):
                pltpu.make_async_copy(
                    emb_hbm.at[lane],
                    rows_v.at[bi, pl.ds(kc * L, L)], rsems[bi]).wait()

        def compute(b, bi):
            r = [recv_v[b, pl.ds(c * L, L)] for c in range(D // L)]
            norm2 = jnp.sum(sum(rc * rc for rc in r))
            inv_norm = _inv_sqrt(norm2)
            biv = jnp.full((L,), bi, jnp.int32)
            bv = jnp.full((L,), b, jnp.int32)

            def dbody(dj, acc):
                out = list(acc)
                for i in range(8):
                    d = dj * 8 + i
                    dvec = (jnp.full((L,), d, jnp.int32) + lane) & 63
                    rd = plsc.load_gather(recv_v, [bv, dvec])
                    for kc in range(4):
                        kidx = lane + kc * L
                        mask = kmask3 if kc == 3 else None
                        vals = plsc.load_gather(
                            rows_v, [biv, kidx, dvec], mask=mask)
                        out[kc] = out[kc] + vals * rd
                return tuple(out)

            zero = jnp.zeros((L,), jnp.float32)
            acc = lax.fori_loop(0, D // 8, dbody, (zero, zero, zero, zero))

            cos = [a * inv_norm for a in acc]
            cos3 = jnp.where(kmask3, cos[3], -1e30)
            m = jnp.max(jnp.maximum(jnp.maximum(cos[0], cos[1]),
                                    jnp.maximum(cos[2], cos3)))
            es = [jnp.exp(c - m) for c in cos[:3]]
            e3 = jnp.where(kmask3, jnp.exp(cos3 - m), 0.0)
            s = jnp.sum(es[0] + es[1] + es[2] + e3)
            logs = _log(jnp.full((L,), s, jnp.float32))
            loss = logs[0] + m - cos[0][0]

            rmax = jnp.max(jnp.maximum(jnp.maximum(r[0], r[1]),
                                       jnp.maximum(r[2], r[3])))
            accv = (r[0][0] >= rmax).astype(jnp.float32)
            return loss, accv

        def gbody(g, carry):
            for bi in range(L):
                hop2_issue(g * L + bi, bi)
            lv = jnp.zeros((L,), jnp.float32)
            av = jnp.zeros((L,), jnp.float32)
            for bi in range(L):
                rows_wait(bi)
                loss, accv = compute(g * L + bi, bi)
                sel = lane == bi
                lv = jnp.where(sel, loss, lv)
                av = jnp.where(sel, accv, av)

            o = pl.multiple_of(g * L, L)
            loss_v[pl.ds(o, L)] = lv
            acc_v[pl.ds(o, L)] = av
            return carry

        lax.fori_loop(0, NG, gbody, 0)

        pltpu.sync_copy(loss_v, loss_hbm.at[pl.ds(base, bpw)])
        pltpu.sync_copy(acc_v, acc_hbm.at[pl.ds(base, bpw)])

    return sc_kernel


def kernel(_sender_input, _message, _receiver_input, receiver_output,
           _labels, _aux_input, train_emb, train_nns, dev_emb, dev_nns):
    B, D = receiver_output.shape
    V, NN = train_nns.shape
    batch_nns = jnp.take(train_nns, _labels, axis=0)
    batch_nns = jnp.concatenate(
        [batch_nns, batch_nns[:, : D - NN]], axis=1)
    sc = _make_sc_kernel(B, V, D)
    loss, acc = sc(batch_nns, receiver_output, train_emb)
    return (loss, acc)


# speedup vs baseline: 3.8647x; 1.0049x over previous
import functools

import jax
import jax.numpy as jnp
from jax import lax
from jax.experimental import pallas as pl
from jax.experimental.pallas import tpu as pltpu
from jax.experimental.pallas import tpu_sc as plsc

_K = 51
_LN2 = 0.6931471805599453


def _inv_sqrt(x):
    bits = lax.bitcast_convert_type(x, jnp.int32)
    y = lax.bitcast_convert_type(
        jnp.int32(0x5F3759DF) - (bits >> 1), jnp.float32)
    for _ in range(3):
        y = y * (1.5 - 0.5 * x * y * y)
    return y


def _log(x):
    bits = lax.bitcast_convert_type(x, jnp.int32)
    e = (bits >> 23) - 127
    m = lax.bitcast_convert_type(
        (bits & jnp.int32(0x007FFFFF)) | jnp.int32(0x3F800000), jnp.float32)
    t = (m - 1.0) / (m + 1.0)
    t2 = t * t
    p = 1.0 + t2 * (1.0 / 3.0 + t2 * (1.0 / 5.0 + t2 * (1.0 / 7.0 + t2 / 9.0)))
    return e.astype(jnp.float32) * _LN2 + 2.0 * t * p


def _make_sc_kernel(B, V, D):
    try:
        info = plsc.get_sparse_core_info()
        NC, NS, L = info.num_cores, info.num_subcores, info.num_lanes
    except ValueError:
        NC, NS, L = 2, 16, 16
    NW = NC * NS
    bpw = B // NW
    NG = bpw // L

    mesh = plsc.VectorSubcoreMesh(
        core_axis_name="c", subcore_axis_name="s",
        num_cores=NC, num_subcores=NS)

    @functools.partial(
        pl.kernel,
        out_type=[
            jax.ShapeDtypeStruct((B,), jnp.float32),
            jax.ShapeDtypeStruct((B,), jnp.float32),
        ],
        mesh=mesh,
        scratch_types=[
            pltpu.VMEM((bpw, D), jnp.int32),
            pltpu.VMEM((bpw, D), jnp.float32),
            pltpu.VMEM((L, 4 * L, D), jnp.float32),
            [pltpu.SemaphoreType.DMA for _ in range(L)],
            pltpu.VMEM((bpw,), jnp.float32),
            pltpu.VMEM((bpw,), jnp.float32),
        ],
        compiler_params=pltpu.CompilerParams(
            needs_layout_passes=False, use_tc_tiling_on_sc=False),
    )
    def sc_kernel(bnns_hbm, recv_hbm, emb_hbm, loss_hbm, acc_hbm,
                  ids_v, recv_v, rows_v, rsems, loss_v, acc_v):
        wid = lax.axis_index("s") * NC + lax.axis_index("c")
        base = wid * bpw
        lane = lax.iota(jnp.int32, L)
        kmask3 = lane < (_K - 3 * L)

        pltpu.sync_copy(bnns_hbm.at[pl.ds(base, bpw)], ids_v)
        pltpu.sync_copy(recv_hbm.at[pl.ds(base, bpw)], recv_v)

        def hop2_issue(b, bi):
            pltpu.make_async_copy(
                emb_hbm.at[ids_v.at[b]], rows_v.at[bi], rsems[bi]).start()

        def rows_wait(bi):
            pltpu.make_async_copy(
                emb_hbm.at[ids_v.at[0]], rows_v.at[bi], rsems[bi]).wait()

        def compute(b, bi):
            r = [recv_v[b, pl.ds(c * L, L)] for c in range(D // L)]
            norm2 = jnp.sum(sum(rc * rc for rc in r))
            inv_norm = _inv_sqrt(norm2)
            biv = jnp.full((L,), bi, jnp.int32)
            bv = jnp.full((L,), b, jnp.int32)

            def dbody(dj, acc):
                out = list(acc)
                for i in range(8):
                    d = dj * 8 + i
                    dvec = (jnp.full((L,), d, jnp.int32) + lane) & 63
                    rd = plsc.load_gather(recv_v, [bv, dvec])
                    for kc in range(4):
                        kidx = lane + kc * L
                        mask = kmask3 if kc == 3 else None
                        vals = plsc.load_gather(
                            rows_v, [biv, kidx, dvec], mask=mask)
                        out[kc] = out[kc] + vals * rd
                return tuple(out)

            zero = jnp.zeros((L,), jnp.float32)
            acc = lax.fori_loop(0, D // 8, dbody, (zero, zero, zero, zero))

            cos = [a * inv_norm for a in acc]
            cos3 = jnp.where(kmask3, cos[3], -1e30)
            m = jnp.max(jnp.maximum(jnp.maximum(cos[0], cos[1]),
                                    jnp.maximum(cos[2], cos3)))
            es = [jnp.exp(c - m) for c in cos[:3]]
            e3 = jnp.where(kmask3, jnp.exp(cos3 - m), 0.0)
            s = jnp.sum(es[0] + es[1] + es[2] + e3)
            logs = _log(jnp.full((L,), s, jnp.float32))
            loss = logs[0] + m - cos[0][0]

            rmax = jnp.max(jnp.maximum(jnp.maximum(r[0], r[1]),
                                       jnp.maximum(r[2], r[3])))
            accv = (r[0][0] >= rmax).astype(jnp.float32)
            return loss, accv

        def gbody(g, carry):
            for bi in range(L):
                hop2_issue(g * L + bi, bi)
            lv = jnp.zeros((L,), jnp.float32)
            av = jnp.zeros((L,), jnp.float32)
            for bi in range(L):
                rows_wait(bi)
                loss, accv = compute(g * L + bi, bi)
                sel = lane == bi
                lv = jnp.where(sel, loss, lv)
                av = jnp.where(sel, accv, av)

            o = pl.multiple_of(g * L, L)
            loss_v[pl.ds(o, L)] = lv
            acc_v[pl.ds(o, L)] = av
            return carry

        lax.fori_loop(0, NG, gbody, 0)

        pltpu.sync_copy(loss_v, loss_hbm.at[pl.ds(base, bpw)])
        pltpu.sync_copy(acc_v, acc_hbm.at[pl.ds(base, bpw)])

    return sc_kernel


def kernel(_sender_input, _message, _receiver_input, receiver_output,
           _labels, _aux_input, train_emb, train_nns, dev_emb, dev_nns):
    B, D = receiver_output.shape
    V, NN = train_nns.shape
    batch_nns = jnp.take(train_nns, _labels, axis=0)
    batch_nns = jnp.concatenate(
        [batch_nns, batch_nns[:, : D - NN]], axis=1)
    sc = _make_sc_kernel(B, V, D)
    loss, acc = sc(batch_nns, receiver_output, train_emb)
    return (loss, acc)
